# baseline (speedup 1.0000x reference)
import math

import jax
import jax.numpy as jnp
from jax.experimental import pallas as pl
from jax.experimental.pallas import tpu as pltpu

D = 256
H = 8
DH = 32
DFF = 1024
T = 2048
NSEG = 8
POS_BASE = 10000.0
BIAS = math.pi / 4

RB = 256
NRB = T // RB
PB = RB // 2


def _fused_kernel(cu_ref, flat_ref, Wq_ref, Wk_ref, Wv_ref, Wo_ref,
                  ln1g_ref, ln1b_ref, ff1_ref, fb1_ref, ff2_ref, fb2_ref,
                  ln2g_ref, ln2b_ref, Uc_ref, bc_ref,
                  Wm1_ref, bm1_ref, Wm2_ref, bm2_ref,
                  out_ref,
                  x_ref, q_ref, k_ref, v_ref, a_ref, y_ref,
                  segr_ref, den_ref, tab_ref,
                  wq16_ref, wk16_ref, wv16_ref, wo16_ref,
                  f116_ref, f216_ref, uct16_ref, ucb16_ref, wm116_ref):
    f32 = jnp.float32
    bf16 = jnp.bfloat16

    scale = 1.0 / math.sqrt(DH)
    wq16_ref[...] = (Wq_ref[...] * scale).astype(bf16)
    wk16_ref[...] = Wk_ref[...].astype(bf16)
    wv16_ref[...] = Wv_ref[...].astype(bf16)
    wo16_ref[...] = Wo_ref[...].astype(bf16)
    f116_ref[...] = ff1_ref[...].astype(bf16)
    f216_ref[...] = ff2_ref[...].astype(bf16)
    uct16_ref[...] = Uc_ref[:D, :].astype(bf16)
    ucb16_ref[...] = Uc_ref[D:, :].astype(bf16)
    wm116_ref[...] = Wm1_ref[...].astype(bf16)

    idx_r = jax.lax.broadcasted_iota(jnp.int32, (T, 1), 0)
    seg_r = jnp.zeros((T, 1), jnp.int32)
    for j in range(1, NSEG):
        seg_r = seg_r + (idx_r >= cu_ref[j]).astype(jnp.int32)
    segr_ref[...] = seg_r

    ln_base = math.log(POS_BASE)

    def _ln(x, g, b):
        m = jnp.mean(x, axis=-1, keepdims=True)
        d = x - m
        v = jnp.mean(d * d, axis=-1, keepdims=True)
        return d * jax.lax.rsqrt(v + 1e-5) * g + b

    def _invdiv(rows):
        col = jax.lax.broadcasted_iota(jnp.int32, (rows, D), 1)
        half_idx = (col // 2).astype(f32)
        inv = jnp.exp(half_idx * (-2.0 / D * ln_base))
        ph = BIAS + jnp.where((col % 2) == 0, 0.0, math.pi / 2)
        return inv, ph

    invA, _ = _invdiv(32)
    invB, phB = _invdiv(64)
    angA = (jax.lax.broadcasted_iota(jnp.int32, (32, D), 0).astype(f32)
            * 64.0 * invA)
    tab_ref[0:32, :] = jnp.sin(angA)
    tab_ref[32:64, :] = jnp.sin(angA + (math.pi / 2))
    angB = (jax.lax.broadcasted_iota(jnp.int32, (64, D), 0).astype(f32)
            * invB + phB)
    tab_ref[64:128, :] = jnp.sin(angB)
    tab_ref[128:192, :] = jnp.sin(angB + (math.pi / 2))
    ioh = jax.lax.broadcasted_iota(jnp.int32, (RB, 32), 1)
    iol = jax.lax.broadcasted_iota(jnp.int32, (RB, 64), 1)

    def p1(i, carry):
        rows = pl.ds(i * RB, RB)
        row_id = i * RB + jax.lax.broadcasted_iota(jnp.int32, (RB, 1), 0)
        start = jnp.zeros((RB, 1), jnp.int32)
        for j in range(1, NSEG):
            c = cu_ref[j]
            start = jnp.maximum(start, jnp.where(row_id >= c, c, 0))
        pos = row_id - start
        oh = (ioh == (pos >> 6)).astype(f32)
        ol = (iol == (pos & 63)).astype(f32)
        sinA = jnp.dot(oh, tab_ref[0:32, :], preferred_element_type=f32)
        cosA = jnp.dot(oh, tab_ref[32:64, :], preferred_element_type=f32)
        sinB = jnp.dot(ol, tab_ref[64:128, :], preferred_element_type=f32)
        cosB = jnp.dot(ol, tab_ref[128:192, :], preferred_element_type=f32)
        pe = sinA * cosB + cosA * sinB
        x = flat_ref[rows, :] + pe
        x_ref[rows, :] = x
        x16 = x.astype(bf16)
        q_ref[rows, :] = jnp.dot(x16, wq16_ref[...],
                                 preferred_element_type=f32).astype(bf16)
        k_ref[rows, :] = jnp.dot(x16, wk16_ref[...],
                                 preferred_element_type=f32).astype(bf16)
        v_ref[rows, :] = jnp.dot(x16, wv16_ref[...],
                                 preferred_element_type=f32).astype(bf16)
        return carry

    jax.lax.fori_loop(0, NRB, p1, 0)

    iota_1rb = jax.lax.broadcasted_iota(jnp.int32, (1, RB), 1)
    Eb = (jax.lax.broadcasted_iota(jnp.int32, (H, D), 1) // DH
          == jax.lax.broadcasted_iota(jnp.int32, (H, D), 0)).astype(f32)
    for i in range(NRB):
        qrows = slice(i * RB, (i + 1) * RB)
        row_id = i * RB + jax.lax.broadcasted_iota(jnp.int32, (RB, 1), 0)
        seg_rb = segr_ref[qrows, :]
        qs = [q_ref[qrows, h * DH:(h + 1) * DH] for h in range(H)]
        a_ref[qrows, :] = jnp.zeros((RB, D), f32)
        den_ref[qrows, :] = jnp.zeros((RB, H), f32)
        st = jnp.int32(0)
        for j in range(1, NSEG):
            c = cu_ref[j]
            st = jnp.where(c <= i * RB, jnp.maximum(st, c), st)
        kb_lo = st // RB

        def p2(kb, carry):
            krows = pl.ds(kb * RB, RB)
            col_id = kb * RB + iota_1rb
            seg_ct = jnp.zeros((1, RB), jnp.int32)
            for j in range(1, NSEG):
                seg_ct = seg_ct + (col_id >= cu_ref[j]).astype(jnp.int32)
            mask = (seg_rb == seg_ct) & (row_id >= col_id)
            bias = jnp.where(mask, 0.0, -1e9)
            kt = k_ref[krows, :]
            vt = v_ref[krows, :]
            dens = []
            nhs = []
            for h in range(H):
                lanes = slice(h * DH, (h + 1) * DH)
                s = jax.lax.dot_general(qs[h], kt[:, lanes],
                                        (((1,), (1,)), ((), ())),
                                        preferred_element_type=f32) + bias
                e = jnp.exp(s)
                nhs.append(jnp.dot(e.astype(bf16), vt[:, lanes],
                                   preferred_element_type=f32))
                dens.append(jnp.sum(e, axis=-1, keepdims=True))
            a_ref[qrows, :] = a_ref[qrows, :] + jnp.concatenate(nhs, axis=1)
            den_ref[qrows, :] = den_ref[qrows, :] + jnp.concatenate(dens, axis=1)
            return carry

        jax.lax.fori_loop(kb_lo, i + 1, p2, 0)
        denb = jnp.dot(den_ref[qrows, :], Eb, preferred_element_type=f32)
        a_ref[qrows, :] = a_ref[qrows, :] / denb

    def p3(i, carry):
        rows = pl.ds(i * RB, RB)
        o = jnp.dot(a_ref[rows, :].astype(bf16), wo16_ref[...],
                    preferred_element_type=f32)
        x1 = _ln(x_ref[rows, :] + o, ln1g_ref[...], ln1b_ref[...])
        fmid = jnp.maximum(jnp.dot(x1.astype(bf16), f116_ref[...],
                                   preferred_element_type=f32)
                           + fb1_ref[...], 0.0)
        f = jnp.dot(fmid.astype(bf16), f216_ref[...],
                    preferred_element_type=f32) + fb2_ref[...]
        y_ref[rows, :] = _ln(x1 + f, ln2g_ref[...], ln2b_ref[...])
        return carry

    jax.lax.fori_loop(0, NRB, p3, 0)

    prow = jax.lax.broadcasted_iota(jnp.int32, (PB, RB), 0)
    pcol = jax.lax.broadcasted_iota(jnp.int32, (PB, RB), 1)
    El = (pcol == 2 * prow).astype(f32)
    Er = (pcol == 2 * prow + 1).astype(f32)

    def p4(i, carry):
        yb = y_ref[pl.ds(i * RB, RB), :]
        h_l = jnp.dot(El, yb, preferred_element_type=f32)
        h_r = jnp.dot(Er, yb, preferred_element_type=f32)
        gates = (jnp.dot(h_l.astype(bf16), uct16_ref[...],
                         preferred_element_type=f32)
                 + jnp.dot(h_r.astype(bf16), ucb16_ref[...],
                           preferred_element_type=f32)
                 + bc_ref[...])
        ig = gates[:, 0 * D:1 * D]
        og = gates[:, 1 * D:2 * D]
        ug = gates[:, 2 * D:3 * D]
        fl = gates[:, 3 * D:4 * D]
        fr = gates[:, 4 * D:5 * D]
        c = (jax.nn.sigmoid(ig) * jnp.tanh(ug)
             + jax.nn.sigmoid(fl) * h_l + jax.nn.sigmoid(fr) * h_r)
        hh = jax.nn.sigmoid(og) * jnp.tanh(c)
        mid = jnp.maximum(jnp.dot(hh.astype(bf16), wm116_ref[...],
                                  preferred_element_type=f32)
                          + bm1_ref[...], 0.0)
        out_ref[pl.ds(i * PB, PB), :] = (
            jnp.dot(mid, Wm2_ref[...], preferred_element_type=f32)
            + bm2_ref[...])
        return carry

    jax.lax.fori_loop(0, NRB, p4, 0)


def kernel(flat, cu_seqlens, Wq, Wk, Wv, Wo, ln1_g, ln1_b, ff1, fb1, ff2, fb2,
           ln2_g, ln2_b, Uc, bc, Wm1, bm1, Wm2, bm2):
    args = (
        cu_seqlens.astype(jnp.int32),
        flat,
        Wq, Wk, Wv, Wo,
        ln1_g.reshape(1, D), ln1_b.reshape(1, D),
        ff1, fb1.reshape(1, DFF), ff2, fb2.reshape(1, D),
        ln2_g.reshape(1, D), ln2_b.reshape(1, D),
        Uc, bc.reshape(1, 5 * D),
        Wm1, bm1.reshape(1, 2 * D), Wm2, bm2.reshape(1, 1),
    )
    in_specs = [pl.BlockSpec(memory_space=pltpu.SMEM)] + [
        pl.BlockSpec(memory_space=pltpu.VMEM)] * (len(args) - 1)
    out = pl.pallas_call(
        _fused_kernel,
        out_shape=jax.ShapeDtypeStruct((T // 2, 1), jnp.float32),
        in_specs=in_specs,
        out_specs=pl.BlockSpec(memory_space=pltpu.VMEM),
        scratch_shapes=[
            pltpu.VMEM((T, D), jnp.float32),
            pltpu.VMEM((T, D), jnp.bfloat16),
            pltpu.VMEM((T, D), jnp.bfloat16),
            pltpu.VMEM((T, D), jnp.bfloat16),
            pltpu.VMEM((T, D), jnp.float32),
            pltpu.VMEM((T, D), jnp.float32),
            pltpu.VMEM((T, 1), jnp.int32),
            pltpu.VMEM((T, H), jnp.float32),
            pltpu.VMEM((192, D), jnp.float32),
            pltpu.VMEM((D, D), jnp.bfloat16),
            pltpu.VMEM((D, D), jnp.bfloat16),
            pltpu.VMEM((D, D), jnp.bfloat16),
            pltpu.VMEM((D, D), jnp.bfloat16),
            pltpu.VMEM((D, DFF), jnp.bfloat16),
            pltpu.VMEM((DFF, D), jnp.bfloat16),
            pltpu.VMEM((D, 5 * D), jnp.bfloat16),
            pltpu.VMEM((D, 5 * D), jnp.bfloat16),
            pltpu.VMEM((D, 2 * D), jnp.bfloat16),
        ],
    )(*args)
    return out


# speedup vs baseline: 1.0132x; 1.0132x over previous
import math

import jax
import jax.numpy as jnp
from jax.experimental import pallas as pl
from jax.experimental.pallas import tpu as pltpu

D = 256
H = 8
DH = 32
DFF = 1024
T = 2048
NSEG = 8
POS_BASE = 10000.0
BIAS = math.pi / 4

RB = 256
NRB = T // RB
PB = RB // 2


def _fused_kernel(cu_ref, flat_ref, Wq_ref, Wk_ref, Wv_ref, Wo_ref,
                  ln1g_ref, ln1b_ref, ff1_ref, fb1_ref, ff2_ref, fb2_ref,
                  ln2g_ref, ln2b_ref, Uc_ref, bc_ref,
                  Wm1_ref, bm1_ref, Wm2_ref, bm2_ref,
                  out_ref,
                  x_ref, q_ref, k_ref, v_ref, a_ref, y_ref,
                  segr_ref, den_ref, tab_ref,
                  wq16_ref, wk16_ref, wv16_ref, wo16_ref,
                  f116_ref, f216_ref, uct16_ref, ucb16_ref, wm116_ref):
    f32 = jnp.float32
    bf16 = jnp.bfloat16

    scale = 1.0 / math.sqrt(DH)
    wq16_ref[...] = (Wq_ref[...] * scale).astype(bf16)
    wk16_ref[...] = Wk_ref[...].astype(bf16)
    wv16_ref[...] = Wv_ref[...].astype(bf16)
    wo16_ref[...] = Wo_ref[...].astype(bf16)
    f116_ref[...] = ff1_ref[...].astype(bf16)
    f216_ref[...] = ff2_ref[...].astype(bf16)
    uct16_ref[...] = Uc_ref[:D, :].astype(bf16)
    ucb16_ref[...] = Uc_ref[D:, :].astype(bf16)
    wm116_ref[...] = Wm1_ref[...].astype(bf16)

    idx_r = jax.lax.broadcasted_iota(jnp.int32, (T, 1), 0)
    seg_r = jnp.zeros((T, 1), jnp.int32)
    for j in range(1, NSEG):
        seg_r = seg_r + (idx_r >= cu_ref[j]).astype(jnp.int32)
    segr_ref[...] = seg_r

    ln_base = math.log(POS_BASE)

    def _ln(x, g, b):
        m = jnp.mean(x, axis=-1, keepdims=True)
        d = x - m
        v = jnp.mean(d * d, axis=-1, keepdims=True)
        return d * jax.lax.rsqrt(v + 1e-5) * g + b

    def _invdiv(rows):
        col = jax.lax.broadcasted_iota(jnp.int32, (rows, D), 1)
        half_idx = (col // 2).astype(f32)
        inv = jnp.exp(half_idx * (-2.0 / D * ln_base))
        ph = BIAS + jnp.where((col % 2) == 0, 0.0, math.pi / 2)
        return inv, ph

    invA, _ = _invdiv(32)
    invB, phB = _invdiv(64)
    angA = (jax.lax.broadcasted_iota(jnp.int32, (32, D), 0).astype(f32)
            * 64.0 * invA)
    tab_ref[0:32, :] = jnp.sin(angA)
    tab_ref[32:64, :] = jnp.sin(angA + (math.pi / 2))
    angB = (jax.lax.broadcasted_iota(jnp.int32, (64, D), 0).astype(f32)
            * invB + phB)
    tab_ref[64:128, :] = jnp.sin(angB)
    tab_ref[128:192, :] = jnp.sin(angB + (math.pi / 2))
    ioh = jax.lax.broadcasted_iota(jnp.int32, (RB, 32), 1)
    iol = jax.lax.broadcasted_iota(jnp.int32, (RB, 64), 1)

    def p1(i, carry):
        rows = pl.ds(i * RB, RB)
        row_id = i * RB + jax.lax.broadcasted_iota(jnp.int32, (RB, 1), 0)
        start = jnp.zeros((RB, 1), jnp.int32)
        for j in range(1, NSEG):
            c = cu_ref[j]
            start = jnp.maximum(start, jnp.where(row_id >= c, c, 0))
        pos = row_id - start
        oh = (ioh == (pos >> 6)).astype(f32)
        ol = (iol == (pos & 63)).astype(f32)
        sinA = jnp.dot(oh, tab_ref[0:32, :], preferred_element_type=f32)
        cosA = jnp.dot(oh, tab_ref[32:64, :], preferred_element_type=f32)
        sinB = jnp.dot(ol, tab_ref[64:128, :], preferred_element_type=f32)
        cosB = jnp.dot(ol, tab_ref[128:192, :], preferred_element_type=f32)
        pe = sinA * cosB + cosA * sinB
        x = flat_ref[rows, :] + pe
        x_ref[rows, :] = x
        x16 = x.astype(bf16)
        q_ref[rows, :] = jnp.dot(x16, wq16_ref[...],
                                 preferred_element_type=f32).astype(bf16)
        k_ref[rows, :] = jnp.dot(x16, wk16_ref[...],
                                 preferred_element_type=f32).astype(bf16)
        v_ref[rows, :] = jnp.dot(x16, wv16_ref[...],
                                 preferred_element_type=f32).astype(bf16)
        return carry

    jax.lax.fori_loop(0, NRB, p1, 0)

    iota_1rb = jax.lax.broadcasted_iota(jnp.int32, (1, RB), 1)
    Eb = (jax.lax.broadcasted_iota(jnp.int32, (H, D), 1) // DH
          == jax.lax.broadcasted_iota(jnp.int32, (H, D), 0)).astype(f32)
    for i in range(NRB):
        qrows = slice(i * RB, (i + 1) * RB)
        row_id = i * RB + jax.lax.broadcasted_iota(jnp.int32, (RB, 1), 0)
        seg_rb = segr_ref[qrows, :]
        qs = [q_ref[qrows, h * DH:(h + 1) * DH] for h in range(H)]
        a_ref[qrows, :] = jnp.zeros((RB, D), f32)
        den_ref[qrows, :] = jnp.zeros((RB, H), f32)
        st = jnp.int32(0)
        for j in range(1, NSEG):
            c = cu_ref[j]
            st = jnp.where(c <= i * RB, jnp.maximum(st, c), st)
        kb_lo = st // RB

        def p2(kb, carry):
            krows = pl.ds(kb * RB, RB)
            col_id = kb * RB + iota_1rb
            seg_ct = jnp.zeros((1, RB), jnp.int32)
            for j in range(1, NSEG):
                seg_ct = seg_ct + (col_id >= cu_ref[j]).astype(jnp.int32)
            mask = (seg_rb == seg_ct) & (row_id >= col_id)
            bias = jnp.where(mask, 0.0, -1e9)
            dens = []
            nhs = []
            for h in range(H):
                lanes = slice(h * DH, (h + 1) * DH)
                s = jax.lax.dot_general(qs[h], k_ref[krows, lanes],
                                        (((1,), (1,)), ((), ())),
                                        preferred_element_type=f32) + bias
                e = jnp.exp(s)
                nhs.append(jnp.dot(e.astype(bf16), v_ref[krows, lanes],
                                   preferred_element_type=f32))
                dens.append(jnp.sum(e, axis=-1, keepdims=True))
            a_ref[qrows, :] = a_ref[qrows, :] + jnp.concatenate(nhs, axis=1)
            den_ref[qrows, :] = den_ref[qrows, :] + jnp.concatenate(dens, axis=1)
            return carry

        jax.lax.fori_loop(kb_lo, i + 1, p2, 0)
        denb = jnp.dot(den_ref[qrows, :], Eb, preferred_element_type=f32)
        a_ref[qrows, :] = a_ref[qrows, :] / denb

    def p3(i, carry):
        rows = pl.ds(i * RB, RB)
        o = jnp.dot(a_ref[rows, :].astype(bf16), wo16_ref[...],
                    preferred_element_type=f32)
        x1 = _ln(x_ref[rows, :] + o, ln1g_ref[...], ln1b_ref[...])
        fmid = jnp.maximum(jnp.dot(x1.astype(bf16), f116_ref[...],
                                   preferred_element_type=f32)
                           + fb1_ref[...], 0.0)
        f = jnp.dot(fmid.astype(bf16), f216_ref[...],
                    preferred_element_type=f32) + fb2_ref[...]
        y_ref[rows, :] = _ln(x1 + f, ln2g_ref[...], ln2b_ref[...])
        return carry

    jax.lax.fori_loop(0, NRB, p3, 0)

    prow = jax.lax.broadcasted_iota(jnp.int32, (PB, RB), 0)
    pcol = jax.lax.broadcasted_iota(jnp.int32, (PB, RB), 1)
    El = (pcol == 2 * prow).astype(f32)
    Er = (pcol == 2 * prow + 1).astype(f32)

    def p4(i, carry):
        yb = y_ref[pl.ds(i * RB, RB), :]
        h_l = jnp.dot(El, yb, preferred_element_type=f32)
        h_r = jnp.dot(Er, yb, preferred_element_type=f32)
        gates = (jnp.dot(h_l.astype(bf16), uct16_ref[...],
                         preferred_element_type=f32)
                 + jnp.dot(h_r.astype(bf16), ucb16_ref[...],
                           preferred_element_type=f32)
                 + bc_ref[...])
        ig = gates[:, 0 * D:1 * D]
        og = gates[:, 1 * D:2 * D]
        ug = gates[:, 2 * D:3 * D]
        fl = gates[:, 3 * D:4 * D]
        fr = gates[:, 4 * D:5 * D]
        c = (jax.nn.sigmoid(ig) * jnp.tanh(ug)
             + jax.nn.sigmoid(fl) * h_l + jax.nn.sigmoid(fr) * h_r)
        hh = jax.nn.sigmoid(og) * jnp.tanh(c)
        mid = jnp.maximum(jnp.dot(hh.astype(bf16), wm116_ref[...],
                                  preferred_element_type=f32)
                          + bm1_ref[...], 0.0)
        out_ref[pl.ds(i * PB, PB), :] = (
            jnp.dot(mid, Wm2_ref[...], preferred_element_type=f32)
            + bm2_ref[...])
        return carry

    jax.lax.fori_loop(0, NRB, p4, 0)


def kernel(flat, cu_seqlens, Wq, Wk, Wv, Wo, ln1_g, ln1_b, ff1, fb1, ff2, fb2,
           ln2_g, ln2_b, Uc, bc, Wm1, bm1, Wm2, bm2):
    args = (
        cu_seqlens.astype(jnp.int32),
        flat,
        Wq, Wk, Wv, Wo,
        ln1_g.reshape(1, D), ln1_b.reshape(1, D),
        ff1, fb1.reshape(1, DFF), ff2, fb2.reshape(1, D),
        ln2_g.reshape(1, D), ln2_b.reshape(1, D),
        Uc, bc.reshape(1, 5 * D),
        Wm1, bm1.reshape(1, 2 * D), Wm2, bm2.reshape(1, 1),
    )
    in_specs = [pl.BlockSpec(memory_space=pltpu.SMEM)] + [
        pl.BlockSpec(memory_space=pltpu.VMEM)] * (len(args) - 1)
    out = pl.pallas_call(
        _fused_kernel,
        out_shape=jax.ShapeDtypeStruct((T // 2, 1), jnp.float32),
        in_specs=in_specs,
        out_specs=pl.BlockSpec(memory_space=pltpu.VMEM),
        scratch_shapes=[
            pltpu.VMEM((T, D), jnp.float32),
            pltpu.VMEM((T, D), jnp.bfloat16),
            pltpu.VMEM((T, D), jnp.bfloat16),
            pltpu.VMEM((T, D), jnp.bfloat16),
            pltpu.VMEM((T, D), jnp.float32),
            pltpu.VMEM((T, D), jnp.float32),
            pltpu.VMEM((T, 1), jnp.int32),
            pltpu.VMEM((T, H), jnp.float32),
            pltpu.VMEM((192, D), jnp.float32),
            pltpu.VMEM((D, D), jnp.bfloat16),
            pltpu.VMEM((D, D), jnp.bfloat16),
            pltpu.VMEM((D, D), jnp.bfloat16),
            pltpu.VMEM((D, D), jnp.bfloat16),
            pltpu.VMEM((D, DFF), jnp.bfloat16),
            pltpu.VMEM((DFF, D), jnp.bfloat16),
            pltpu.VMEM((D, 5 * D), jnp.bfloat16),
            pltpu.VMEM((D, 5 * D), jnp.bfloat16),
            pltpu.VMEM((D, 2 * D), jnp.bfloat16),
        ],
    )(*args)
    return out
